# baseline (device time: 292168 ns/iter reference)
import functools

import jax
import jax.numpy as jnp
from jax import lax
from jax.experimental import pallas as pl
from jax.experimental.pallas import tpu as pltpu

N_DEV = 16
B = 2
S = 1024
D = 1024
DH = 128
H_LOC = 8
EPS = 1e-5
SCALE = 0.08838834764831843


def _ln_mod_body(x_ref, sc_ref, sh_ref, o_ref):
    for b in range(B):
        xb = x_ref[b]
        m = jnp.mean(xb, axis=-1, keepdims=True)
        v = jnp.mean((xb - m) * (xb - m), axis=-1, keepdims=True)
        xn = (xb - m) * lax.rsqrt(v + EPS)
        o_ref[b] = xn * (1.0 + sc_ref[b : b + 1, :]) + sh_ref[b : b + 1, :]


def _ln_mod(x, sc, sh):
    return pl.pallas_call(
        _ln_mod_body,
        out_shape=jax.ShapeDtypeStruct((B, S, D), jnp.float32),
        in_specs=[pl.BlockSpec(memory_space=pltpu.VMEM)] * 3,
        out_specs=pl.BlockSpec(memory_space=pltpu.VMEM),
    )(x, sc, sh)


def _resid_body(x_ref, g_ref, y_ref, o_ref):
    for b in range(B):
        o_ref[b] = x_ref[b] + g_ref[b : b + 1, :] * y_ref[b]


def _resid(x, g, y):
    return pl.pallas_call(
        _resid_body,
        out_shape=jax.ShapeDtypeStruct((B, S, D), jnp.float32),
        in_specs=[pl.BlockSpec(memory_space=pltpu.VMEM)] * 3,
        out_specs=pl.BlockSpec(memory_space=pltpu.VMEM),
    )(x, g, y)


def _attn_body(xn_ref, wq_ref, wk_ref, wv_ref, wo_ref, o_ref):
    h = pl.program_id(1)
    bf = jnp.bfloat16
    xb = xn_ref[0].astype(bf)
    q = jnp.dot(
        xb, wq_ref[...].astype(bf), preferred_element_type=jnp.float32
    ).astype(bf)
    k = jnp.dot(
        xb, wk_ref[...].astype(bf), preferred_element_type=jnp.float32
    ).astype(bf)
    v = jnp.dot(
        xb, wv_ref[...].astype(bf), preferred_element_type=jnp.float32
    ).astype(bf)
    s = (
        lax.dot_general(
            q, k, (((1,), (1,)), ((), ())), preferred_element_type=jnp.float32
        )
        * SCALE
    )
    m = jnp.max(s, axis=-1, keepdims=True)
    p = jnp.exp(s - m)
    l = jnp.sum(p, axis=-1, keepdims=True)
    o = jnp.dot(p.astype(bf), v, preferred_element_type=jnp.float32) / l
    contrib = jnp.dot(
        o.astype(bf), wo_ref[...].astype(bf), preferred_element_type=jnp.float32
    )

    @pl.when(h == 0)
    def _():
        o_ref[0] = contrib

    @pl.when(h != 0)
    def _():
        o_ref[0] = o_ref[0] + contrib


def _attn_partial(xn, wq, wk, wv, wo):
    return pl.pallas_call(
        _attn_body,
        grid=(B, H_LOC),
        in_specs=[
            pl.BlockSpec((1, S, D), lambda b, h: (b, 0, 0)),
            pl.BlockSpec((D, DH), lambda b, h: (0, h)),
            pl.BlockSpec((D, DH), lambda b, h: (0, h)),
            pl.BlockSpec((D, DH), lambda b, h: (0, h)),
            pl.BlockSpec((DH, D), lambda b, h: (h, 0)),
        ],
        out_specs=pl.BlockSpec((1, S, D), lambda b, h: (b, 0, 0)),
        out_shape=jax.ShapeDtypeStruct((B, S, D), jnp.float32),
    )(xn, wq, wk, wv, wo)


def _mlp_body(xn_ref, w1_ref, w2_ref, o_ref):
    bf = jnp.bfloat16
    for b in range(B):
        hh = jnp.dot(
            xn_ref[b].astype(bf), w1_ref[...].astype(bf),
            preferred_element_type=jnp.float32,
        )
        hh = hh * jax.nn.sigmoid(hh)
        o_ref[b] = jnp.dot(
            hh.astype(bf), w2_ref[...].astype(bf),
            preferred_element_type=jnp.float32,
        )


def _mlp_partial(xn, w1, w2):
    return pl.pallas_call(
        _mlp_body,
        out_shape=jax.ShapeDtypeStruct((B, S, D), jnp.float32),
        in_specs=[pl.BlockSpec(memory_space=pltpu.VMEM)] * 3,
        out_specs=pl.BlockSpec(memory_space=pltpu.VMEM),
    )(xn, w1, w2)


def _allreduce_body(
    x_ref,
    o_ref,
    comm_r,
    comm_l,
    stg_r,
    stg_l,
    ag_comm_r,
    ag_comm_l,
    ag_stg_r,
    ag_stg_l,
    rs_send_r,
    rs_recv_r,
    ag_send_r,
    ag_recv_r,
    rs_send_l,
    rs_recv_l,
    ag_send_l,
    ag_recv_l,
):
    my = lax.axis_index("i")
    left = lax.rem(my - 1 + N_DEV, N_DEV)
    right = lax.rem(my + 1, N_DEV)
    rows = o_ref.shape[0]
    half = rows // 2
    C = half // N_DEV

    def sl_r(idx):
        return pl.ds(idx * C, C)

    def sl_l(idx):
        return pl.ds(half + idx * C, C)

    barrier = pltpu.get_barrier_semaphore()
    for nbr in (left, right):
        pl.semaphore_signal(
            barrier, inc=1, device_id=(nbr,), device_id_type=pl.DeviceIdType.MESH
        )
    pl.semaphore_wait(barrier, 2)

    o_ref[...] = x_ref[...]

    pend = []
    for s in range(N_DEV - 1):
        si_r = lax.rem(my - s + N_DEV, N_DEV)
        stg_r[s] = o_ref[sl_r(si_r)].astype(jnp.bfloat16)
        rd_r = pltpu.make_async_remote_copy(
            src_ref=stg_r.at[s],
            dst_ref=comm_r.at[s],
            send_sem=rs_send_r.at[s],
            recv_sem=rs_recv_r.at[s],
            device_id=(right,),
            device_id_type=pl.DeviceIdType.MESH,
        )
        si_l = lax.rem(my + s, N_DEV)
        stg_l[s] = o_ref[sl_l(si_l)].astype(jnp.bfloat16)
        rd_l = pltpu.make_async_remote_copy(
            src_ref=stg_l.at[s],
            dst_ref=comm_l.at[s],
            send_sem=rs_send_l.at[s],
            recv_sem=rs_recv_l.at[s],
            device_id=(left,),
            device_id_type=pl.DeviceIdType.MESH,
        )
        rd_r.start()
        rd_l.start()
        rd_r.wait_recv()
        ri_r = sl_r(lax.rem(my - s - 1 + N_DEV, N_DEV))
        o_ref[ri_r] = o_ref[ri_r] + comm_r[s].astype(jnp.float32)
        rd_l.wait_recv()
        ri_l = sl_l(lax.rem(my + s + 1, N_DEV))
        o_ref[ri_l] = o_ref[ri_l] + comm_l[s].astype(jnp.float32)
        pend += [rd_r, rd_l]
    for rd in pend:
        rd.wait_send()

    ag_stg_r[...] = o_ref[sl_r(lax.rem(my + 1, N_DEV))].astype(jnp.bfloat16)
    ag_stg_l[...] = o_ref[sl_l(lax.rem(my - 1 + N_DEV, N_DEV))].astype(jnp.bfloat16)
    pend = []
    for s in range(N_DEV - 1):
        src_r = ag_stg_r if s == 0 else ag_comm_r.at[s - 1]
        rd_r = pltpu.make_async_remote_copy(
            src_ref=src_r,
            dst_ref=ag_comm_r.at[s],
            send_sem=ag_send_r.at[s],
            recv_sem=ag_recv_r.at[s],
            device_id=(right,),
            device_id_type=pl.DeviceIdType.MESH,
        )
        src_l = ag_stg_l if s == 0 else ag_comm_l.at[s - 1]
        rd_l = pltpu.make_async_remote_copy(
            src_ref=src_l,
            dst_ref=ag_comm_l.at[s],
            send_sem=ag_send_l.at[s],
            recv_sem=ag_recv_l.at[s],
            device_id=(left,),
            device_id_type=pl.DeviceIdType.MESH,
        )
        rd_r.start()
        rd_l.start()
        rd_r.wait_recv()
        o_ref[sl_r(lax.rem(my - s + N_DEV, N_DEV))] = ag_comm_r[s].astype(
            jnp.float32
        )
        rd_l.wait_recv()
        o_ref[sl_l(lax.rem(my + s, N_DEV))] = ag_comm_l[s].astype(jnp.float32)
        pend += [rd_r, rd_l]
    for rd in pend:
        rd.wait_send()


def _ring_allreduce_ring(x2d, cid):
    rows, cols = x2d.shape
    C = rows // 2 // N_DEV
    bufs = [pltpu.VMEM((N_DEV - 1, C, cols), jnp.bfloat16) for _ in range(6)]
    bufs += [pltpu.VMEM((C, cols), jnp.bfloat16) for _ in range(2)]
    return pl.pallas_call(
        _allreduce_body,
        out_shape=jax.ShapeDtypeStruct((rows, cols), jnp.float32),
        in_specs=[pl.BlockSpec(memory_space=pltpu.VMEM)],
        out_specs=pl.BlockSpec(memory_space=pltpu.VMEM),
        scratch_shapes=bufs
        + [pltpu.SemaphoreType.DMA((N_DEV - 1,)) for _ in range(8)],
        compiler_params=pltpu.CompilerParams(collective_id=cid),
    )(x2d)


_P = 4
_Z = 4


def _hier_allreduce_body(
    x_ref,
    o_ref,
    stgA_r,
    comA_r,
    stgA_l,
    comA_l,
    stg1T,
    stg1B,
    com1T,
    com1B,
    stg2T,
    stg2B,
    com2T,
    com2B,
    stg3T,
    stg3B,
    com3T,
    com3B,
    stg4T,
    stg4B,
    com4T,
    com4B,
    stgC_r,
    comC_r,
    stgC_l,
    comC_l,
    semA_send_r,
    semA_recv_r,
    semA_send_l,
    semA_recv_l,
    semB_send,
    semB_recv,
    semC_send_r,
    semC_recv_r,
    semC_send_l,
    semC_recv_l,
):
    my = lax.axis_index("i")
    p = lax.rem(my, _P)
    z = lax.div(my, _P)
    bit0 = lax.rem(z, 2)
    bit1 = lax.div(z, 2)
    right = z * _P + lax.rem(p + 1, _P)
    left = z * _P + lax.rem(p - 1 + _P, _P)
    q1 = my + (1 - 2 * bit0) * _P
    q2 = my + (1 - 2 * bit1) * 2 * _P

    rows = o_ref.shape[0]
    half = rows // 2
    PC = half // _P

    def dev(d):
        return dict(device_id=(d,), device_id_type=pl.DeviceIdType.MESH)

    barrier = pltpu.get_barrier_semaphore()
    for nbr in (left, right, q1, q2):
        pl.semaphore_signal(barrier, inc=1, **dev(nbr))
    pl.semaphore_wait(barrier, 4)

    o_ref[...] = x_ref[...]

    pend = []
    for s in range(_P - 1):
        si_r = lax.rem(p - s + _P, _P)
        stgA_r[s] = o_ref[pl.ds(si_r * PC, PC)].astype(jnp.bfloat16)
        rd_r = pltpu.make_async_remote_copy(
            src_ref=stgA_r.at[s],
            dst_ref=comA_r.at[s],
            send_sem=semA_send_r.at[s],
            recv_sem=semA_recv_r.at[s],
            **dev(right),
        )
        si_l = lax.rem(p + s, _P)
        stgA_l[s] = o_ref[pl.ds(half + si_l * PC, PC)].astype(jnp.bfloat16)
        rd_l = pltpu.make_async_remote_copy(
            src_ref=stgA_l.at[s],
            dst_ref=comA_l.at[s],
            send_sem=semA_send_l.at[s],
            recv_sem=semA_recv_l.at[s],
            **dev(left),
        )
        rd_r.start()
        rd_l.start()
        rd_r.wait_recv()
        ri = pl.ds(lax.rem(p - s - 1 + _P, _P) * PC, PC)
        o_ref[ri] = o_ref[ri] + comA_r[s].astype(jnp.float32)
        rd_l.wait_recv()
        li = pl.ds(half + lax.rem(p + s + 1, _P) * PC, PC)
        o_ref[li] = o_ref[li] + comA_l[s].astype(jnp.float32)
        pend += [rd_r, rd_l]
    for rd in pend:
        rd.wait_send()

    toff = lax.rem(p + 1, _P) * PC
    boff = half + lax.rem(p - 1 + _P, _P) * PC

    HC = PC // 2
    QC = PC // 4

    def exchange(partner, idx, stgT, stgB_, comT, comB_, t_send, b_send, t_keep, b_keep, size):
        stgT[...] = o_ref[pl.ds(t_send, size)].astype(jnp.bfloat16)
        stgB_[...] = o_ref[pl.ds(b_send, size)].astype(jnp.bfloat16)
        rdt = pltpu.make_async_remote_copy(
            src_ref=stgT,
            dst_ref=comT,
            send_sem=semB_send.at[2 * idx],
            recv_sem=semB_recv.at[2 * idx],
            **dev(partner),
        )
        rdb = pltpu.make_async_remote_copy(
            src_ref=stgB_,
            dst_ref=comB_,
            send_sem=semB_send.at[2 * idx + 1],
            recv_sem=semB_recv.at[2 * idx + 1],
            **dev(partner),
        )
        rdt.start()
        rdb.start()
        return rdt, rdb

    kt1 = toff + bit0 * HC
    kb1 = boff + bit0 * HC
    rdt, rdb = exchange(
        q1, 0, stg1T, stg1B, com1T, com1B,
        toff + (1 - bit0) * HC, boff + (1 - bit0) * HC, kt1, kb1, HC,
    )
    rdt.wait_recv()
    sl = pl.ds(kt1, HC)
    o_ref[sl] = o_ref[sl] + com1T[...].astype(jnp.float32)
    rdb.wait_recv()
    sl = pl.ds(kb1, HC)
    o_ref[sl] = o_ref[sl] + com1B[...].astype(jnp.float32)
    rdt.wait_send()
    rdb.wait_send()

    kt2 = kt1 + bit1 * QC
    kb2 = kb1 + bit1 * QC
    rdt, rdb = exchange(
        q2, 1, stg2T, stg2B, com2T, com2B,
        kt1 + (1 - bit1) * QC, kb1 + (1 - bit1) * QC, kt2, kb2, QC,
    )
    rdt.wait_recv()
    sl = pl.ds(kt2, QC)
    o_ref[sl] = o_ref[sl] + com2T[...].astype(jnp.float32)
    rdb.wait_recv()
    sl = pl.ds(kb2, QC)
    o_ref[sl] = o_ref[sl] + com2B[...].astype(jnp.float32)
    rdt.wait_send()
    rdb.wait_send()

    rdt, rdb = exchange(
        q2, 2, stg3T, stg3B, com3T, com3B, kt2, kb2, 0, 0, QC,
    )
    rdt.wait_recv()
    o_ref[pl.ds(kt1 + (1 - bit1) * QC, QC)] = com3T[...].astype(jnp.float32)
    rdb.wait_recv()
    o_ref[pl.ds(kb1 + (1 - bit1) * QC, QC)] = com3B[...].astype(jnp.float32)
    rdt.wait_send()
    rdb.wait_send()

    rdt, rdb = exchange(
        q1, 3, stg4T, stg4B, com4T, com4B, kt1, kb1, 0, 0, HC,
    )
    rdt.wait_recv()
    o_ref[pl.ds(toff + (1 - bit0) * HC, HC)] = com4T[...].astype(jnp.float32)
    rdb.wait_recv()
    o_ref[pl.ds(boff + (1 - bit0) * HC, HC)] = com4B[...].astype(jnp.float32)
    rdt.wait_send()
    rdb.wait_send()

    stgC_r[...] = o_ref[pl.ds(toff, PC)].astype(jnp.bfloat16)
    stgC_l[...] = o_ref[pl.ds(boff, PC)].astype(jnp.bfloat16)
    pend = []
    for s in range(_P - 1):
        src_r = stgC_r if s == 0 else comC_r.at[s - 1]
        rd_r = pltpu.make_async_remote_copy(
            src_ref=src_r,
            dst_ref=comC_r.at[s],
            send_sem=semC_send_r.at[s],
            recv_sem=semC_recv_r.at[s],
            **dev(right),
        )
        src_l = stgC_l if s == 0 else comC_l.at[s - 1]
        rd_l = pltpu.make_async_remote_copy(
            src_ref=src_l,
            dst_ref=comC_l.at[s],
            send_sem=semC_send_l.at[s],
            recv_sem=semC_recv_l.at[s],
            **dev(left),
        )
        rd_r.start()
        rd_l.start()
        rd_r.wait_recv()
        o_ref[pl.ds(lax.rem(p - s + _P, _P) * PC, PC)] = comC_r[s].astype(
            jnp.float32
        )
        rd_l.wait_recv()
        o_ref[pl.ds(half + lax.rem(p + s, _P) * PC, PC)] = comC_l[s].astype(
            jnp.float32
        )
        pend += [rd_r, rd_l]
    for rd in pend:
        rd.wait_send()


def _ring_allreduce(x2d, cid):
    rows, cols = x2d.shape
    PC = rows // 2 // _P
    HC, QC = PC // 2, PC // 4
    bf = jnp.bfloat16
    bufs = [
        pltpu.VMEM((_P - 1, PC, cols), bf),
        pltpu.VMEM((_P - 1, PC, cols), bf),
        pltpu.VMEM((_P - 1, PC, cols), bf),
        pltpu.VMEM((_P - 1, PC, cols), bf),
        pltpu.VMEM((HC, cols), bf),
        pltpu.VMEM((HC, cols), bf),
        pltpu.VMEM((HC, cols), bf),
        pltpu.VMEM((HC, cols), bf),
        pltpu.VMEM((QC, cols), bf),
        pltpu.VMEM((QC, cols), bf),
        pltpu.VMEM((QC, cols), bf),
        pltpu.VMEM((QC, cols), bf),
        pltpu.VMEM((QC, cols), bf),
        pltpu.VMEM((QC, cols), bf),
        pltpu.VMEM((QC, cols), bf),
        pltpu.VMEM((QC, cols), bf),
        pltpu.VMEM((HC, cols), bf),
        pltpu.VMEM((HC, cols), bf),
        pltpu.VMEM((HC, cols), bf),
        pltpu.VMEM((HC, cols), bf),
        pltpu.VMEM((PC, cols), bf),
        pltpu.VMEM((_P - 1, PC, cols), bf),
        pltpu.VMEM((PC, cols), bf),
        pltpu.VMEM((_P - 1, PC, cols), bf),
    ]
    sems = [
        pltpu.SemaphoreType.DMA((_P - 1,)),
        pltpu.SemaphoreType.DMA((_P - 1,)),
        pltpu.SemaphoreType.DMA((_P - 1,)),
        pltpu.SemaphoreType.DMA((_P - 1,)),
        pltpu.SemaphoreType.DMA((8,)),
        pltpu.SemaphoreType.DMA((8,)),
        pltpu.SemaphoreType.DMA((_P - 1,)),
        pltpu.SemaphoreType.DMA((_P - 1,)),
        pltpu.SemaphoreType.DMA((_P - 1,)),
        pltpu.SemaphoreType.DMA((_P - 1,)),
    ]
    return pl.pallas_call(
        _hier_allreduce_body,
        out_shape=jax.ShapeDtypeStruct((rows, cols), jnp.float32),
        in_specs=[pl.BlockSpec(memory_space=pltpu.VMEM)],
        out_specs=pl.BlockSpec(memory_space=pltpu.VMEM),
        scratch_shapes=bufs + sems,
        compiler_params=pltpu.CompilerParams(collective_id=cid),
    )(x2d)


def kernel(x, Wq, Wk, Wv, Wo, t_emb, W_mod, W_ff1, W_ff2):
    mod = jnp.dot(t_emb, W_mod)
    sa, sha, ga, sm, shm, gm = jnp.split(mod, 6, axis=-1)

    xn1 = _ln_mod(x, sa, sha)
    attn_p = _attn_partial(xn1, Wq, Wk, Wv, Wo)
    attn_sum = _ring_allreduce(attn_p.reshape(B * S, D), 0).reshape(B, S, D)
    x1 = _resid(x, ga, attn_sum)

    xn2 = _ln_mod(x1, sm, shm)
    mlp_p = _mlp_partial(xn2, W_ff1, W_ff2)
    mlp_sum = _ring_allreduce(mlp_p.reshape(B * S, D), 1).reshape(B, S, D)
    return _resid(x1, gm, mlp_sum)


# device time: 284284 ns/iter; 1.0277x vs baseline; 1.0277x over previous
import functools

import jax
import jax.numpy as jnp
from jax import lax
from jax.experimental import pallas as pl
from jax.experimental.pallas import tpu as pltpu

N_DEV = 16
B = 2
S = 1024
D = 1024
DH = 128
H_LOC = 8
EPS = 1e-5
SCALE = 0.08838834764831843


def _ln_mod_body(x_ref, sc_ref, sh_ref, o_ref):
    for b in range(B):
        xb = x_ref[b]
        m = jnp.mean(xb, axis=-1, keepdims=True)
        v = jnp.mean((xb - m) * (xb - m), axis=-1, keepdims=True)
        xn = (xb - m) * lax.rsqrt(v + EPS)
        o_ref[b] = xn * (1.0 + sc_ref[b : b + 1, :]) + sh_ref[b : b + 1, :]


def _ln_mod(x, sc, sh):
    return pl.pallas_call(
        _ln_mod_body,
        out_shape=jax.ShapeDtypeStruct((B, S, D), jnp.float32),
        in_specs=[pl.BlockSpec(memory_space=pltpu.VMEM)] * 3,
        out_specs=pl.BlockSpec(memory_space=pltpu.VMEM),
    )(x, sc, sh)


def _resid_body(x_ref, g_ref, y_ref, o_ref):
    for b in range(B):
        o_ref[b] = x_ref[b] + g_ref[b : b + 1, :] * y_ref[b]


def _resid(x, g, y):
    return pl.pallas_call(
        _resid_body,
        out_shape=jax.ShapeDtypeStruct((B, S, D), jnp.float32),
        in_specs=[pl.BlockSpec(memory_space=pltpu.VMEM)] * 3,
        out_specs=pl.BlockSpec(memory_space=pltpu.VMEM),
    )(x, g, y)


def _attn_body(xn_ref, wq_ref, wk_ref, wv_ref, wo_ref, o_ref):
    h = pl.program_id(1)
    bf = jnp.bfloat16
    xb = xn_ref[0].astype(bf)
    q = jnp.dot(
        xb, wq_ref[...].astype(bf), preferred_element_type=jnp.float32
    ).astype(bf)
    k = jnp.dot(
        xb, wk_ref[...].astype(bf), preferred_element_type=jnp.float32
    ).astype(bf)
    v = jnp.dot(
        xb, wv_ref[...].astype(bf), preferred_element_type=jnp.float32
    ).astype(bf)
    s = (
        lax.dot_general(
            q, k, (((1,), (1,)), ((), ())), preferred_element_type=jnp.float32
        )
        * SCALE
    )
    p = jnp.exp(s)
    l = jnp.sum(p, axis=-1, keepdims=True)
    o = jnp.dot(p.astype(bf), v, preferred_element_type=jnp.float32) / l
    contrib = jnp.dot(
        o.astype(bf), wo_ref[...].astype(bf), preferred_element_type=jnp.float32
    )

    @pl.when(h == 0)
    def _():
        o_ref[0] = contrib

    @pl.when(h != 0)
    def _():
        o_ref[0] = o_ref[0] + contrib


def _attn_partial(xn, wq, wk, wv, wo):
    return pl.pallas_call(
        _attn_body,
        grid=(B, H_LOC),
        in_specs=[
            pl.BlockSpec((1, S, D), lambda b, h: (b, 0, 0)),
            pl.BlockSpec((D, DH), lambda b, h: (0, h)),
            pl.BlockSpec((D, DH), lambda b, h: (0, h)),
            pl.BlockSpec((D, DH), lambda b, h: (0, h)),
            pl.BlockSpec((DH, D), lambda b, h: (h, 0)),
        ],
        out_specs=pl.BlockSpec((1, S, D), lambda b, h: (b, 0, 0)),
        out_shape=jax.ShapeDtypeStruct((B, S, D), jnp.float32),
    )(xn, wq, wk, wv, wo)


def _mlp_body(xn_ref, w1_ref, w2_ref, o_ref):
    bf = jnp.bfloat16
    for b in range(B):
        hh = jnp.dot(
            xn_ref[b].astype(bf), w1_ref[...].astype(bf),
            preferred_element_type=jnp.float32,
        )
        hh = hh * jax.nn.sigmoid(hh)
        o_ref[b] = jnp.dot(
            hh.astype(bf), w2_ref[...].astype(bf),
            preferred_element_type=jnp.float32,
        )


def _mlp_partial(xn, w1, w2):
    return pl.pallas_call(
        _mlp_body,
        out_shape=jax.ShapeDtypeStruct((B, S, D), jnp.float32),
        in_specs=[pl.BlockSpec(memory_space=pltpu.VMEM)] * 3,
        out_specs=pl.BlockSpec(memory_space=pltpu.VMEM),
    )(xn, w1, w2)


def _allreduce_body(
    x_ref,
    o_ref,
    comm_r,
    comm_l,
    stg_r,
    stg_l,
    ag_comm_r,
    ag_comm_l,
    ag_stg_r,
    ag_stg_l,
    rs_send_r,
    rs_recv_r,
    ag_send_r,
    ag_recv_r,
    rs_send_l,
    rs_recv_l,
    ag_send_l,
    ag_recv_l,
):
    my = lax.axis_index("i")
    left = lax.rem(my - 1 + N_DEV, N_DEV)
    right = lax.rem(my + 1, N_DEV)
    rows = o_ref.shape[0]
    half = rows // 2
    C = half // N_DEV

    def sl_r(idx):
        return pl.ds(idx * C, C)

    def sl_l(idx):
        return pl.ds(half + idx * C, C)

    barrier = pltpu.get_barrier_semaphore()
    for nbr in (left, right):
        pl.semaphore_signal(
            barrier, inc=1, device_id=(nbr,), device_id_type=pl.DeviceIdType.MESH
        )
    pl.semaphore_wait(barrier, 2)

    o_ref[...] = x_ref[...]

    pend = []
    for s in range(N_DEV - 1):
        si_r = lax.rem(my - s + N_DEV, N_DEV)
        stg_r[s] = o_ref[sl_r(si_r)].astype(jnp.bfloat16)
        rd_r = pltpu.make_async_remote_copy(
            src_ref=stg_r.at[s],
            dst_ref=comm_r.at[s],
            send_sem=rs_send_r.at[s],
            recv_sem=rs_recv_r.at[s],
            device_id=(right,),
            device_id_type=pl.DeviceIdType.MESH,
        )
        si_l = lax.rem(my + s, N_DEV)
        stg_l[s] = o_ref[sl_l(si_l)].astype(jnp.bfloat16)
        rd_l = pltpu.make_async_remote_copy(
            src_ref=stg_l.at[s],
            dst_ref=comm_l.at[s],
            send_sem=rs_send_l.at[s],
            recv_sem=rs_recv_l.at[s],
            device_id=(left,),
            device_id_type=pl.DeviceIdType.MESH,
        )
        rd_r.start()
        rd_l.start()
        rd_r.wait_recv()
        ri_r = sl_r(lax.rem(my - s - 1 + N_DEV, N_DEV))
        o_ref[ri_r] = o_ref[ri_r] + comm_r[s].astype(jnp.float32)
        rd_l.wait_recv()
        ri_l = sl_l(lax.rem(my + s + 1, N_DEV))
        o_ref[ri_l] = o_ref[ri_l] + comm_l[s].astype(jnp.float32)
        pend += [rd_r, rd_l]
    for rd in pend:
        rd.wait_send()

    ag_stg_r[...] = o_ref[sl_r(lax.rem(my + 1, N_DEV))].astype(jnp.bfloat16)
    ag_stg_l[...] = o_ref[sl_l(lax.rem(my - 1 + N_DEV, N_DEV))].astype(jnp.bfloat16)
    pend = []
    for s in range(N_DEV - 1):
        src_r = ag_stg_r if s == 0 else ag_comm_r.at[s - 1]
        rd_r = pltpu.make_async_remote_copy(
            src_ref=src_r,
            dst_ref=ag_comm_r.at[s],
            send_sem=ag_send_r.at[s],
            recv_sem=ag_recv_r.at[s],
            device_id=(right,),
            device_id_type=pl.DeviceIdType.MESH,
        )
        src_l = ag_stg_l if s == 0 else ag_comm_l.at[s - 1]
        rd_l = pltpu.make_async_remote_copy(
            src_ref=src_l,
            dst_ref=ag_comm_l.at[s],
            send_sem=ag_send_l.at[s],
            recv_sem=ag_recv_l.at[s],
            device_id=(left,),
            device_id_type=pl.DeviceIdType.MESH,
        )
        rd_r.start()
        rd_l.start()
        rd_r.wait_recv()
        o_ref[sl_r(lax.rem(my - s + N_DEV, N_DEV))] = ag_comm_r[s].astype(
            jnp.float32
        )
        rd_l.wait_recv()
        o_ref[sl_l(lax.rem(my + s, N_DEV))] = ag_comm_l[s].astype(jnp.float32)
        pend += [rd_r, rd_l]
    for rd in pend:
        rd.wait_send()


def _ring_allreduce_ring(x2d, cid):
    rows, cols = x2d.shape
    C = rows // 2 // N_DEV
    bufs = [pltpu.VMEM((N_DEV - 1, C, cols), jnp.bfloat16) for _ in range(6)]
    bufs += [pltpu.VMEM((C, cols), jnp.bfloat16) for _ in range(2)]
    return pl.pallas_call(
        _allreduce_body,
        out_shape=jax.ShapeDtypeStruct((rows, cols), jnp.float32),
        in_specs=[pl.BlockSpec(memory_space=pltpu.VMEM)],
        out_specs=pl.BlockSpec(memory_space=pltpu.VMEM),
        scratch_shapes=bufs
        + [pltpu.SemaphoreType.DMA((N_DEV - 1,)) for _ in range(8)],
        compiler_params=pltpu.CompilerParams(collective_id=cid),
    )(x2d)


_P = 4
_Z = 4


def _hier_allreduce_body(
    x_ref,
    o_ref,
    stgA_r,
    comA_r,
    stgA_l,
    comA_l,
    stg1T,
    stg1B,
    com1T,
    com1B,
    stg2T,
    stg2B,
    com2T,
    com2B,
    stg3T,
    stg3B,
    com3T,
    com3B,
    stg4T,
    stg4B,
    com4T,
    com4B,
    stgC_r,
    comC_r,
    stgC_l,
    comC_l,
    semA_send_r,
    semA_recv_r,
    semA_send_l,
    semA_recv_l,
    semB_send,
    semB_recv,
    semC_send_r,
    semC_recv_r,
    semC_send_l,
    semC_recv_l,
):
    my = lax.axis_index("i")
    p = lax.rem(my, _P)
    z = lax.div(my, _P)
    bit0 = lax.rem(z, 2)
    bit1 = lax.div(z, 2)
    right = z * _P + lax.rem(p + 1, _P)
    left = z * _P + lax.rem(p - 1 + _P, _P)
    q1 = my + (1 - 2 * bit0) * _P
    q2 = my + (1 - 2 * bit1) * 2 * _P

    rows = o_ref.shape[0]
    half = rows // 2
    PC = half // _P

    def dev(d):
        return dict(device_id=(d,), device_id_type=pl.DeviceIdType.MESH)

    barrier = pltpu.get_barrier_semaphore()
    for nbr in (left, right, q1, q2):
        pl.semaphore_signal(barrier, inc=1, **dev(nbr))
    pl.semaphore_wait(barrier, 4)

    o_ref[...] = x_ref[...]

    pend = []
    for s in range(_P - 1):
        si_r = lax.rem(p - s + _P, _P)
        stgA_r[s] = o_ref[pl.ds(si_r * PC, PC)].astype(jnp.bfloat16)
        rd_r = pltpu.make_async_remote_copy(
            src_ref=stgA_r.at[s],
            dst_ref=comA_r.at[s],
            send_sem=semA_send_r.at[s],
            recv_sem=semA_recv_r.at[s],
            **dev(right),
        )
        si_l = lax.rem(p + s, _P)
        stgA_l[s] = o_ref[pl.ds(half + si_l * PC, PC)].astype(jnp.bfloat16)
        rd_l = pltpu.make_async_remote_copy(
            src_ref=stgA_l.at[s],
            dst_ref=comA_l.at[s],
            send_sem=semA_send_l.at[s],
            recv_sem=semA_recv_l.at[s],
            **dev(left),
        )
        rd_r.start()
        rd_l.start()
        rd_r.wait_recv()
        ri = pl.ds(lax.rem(p - s - 1 + _P, _P) * PC, PC)
        o_ref[ri] = o_ref[ri] + comA_r[s].astype(jnp.float32)
        rd_l.wait_recv()
        li = pl.ds(half + lax.rem(p + s + 1, _P) * PC, PC)
        o_ref[li] = o_ref[li] + comA_l[s].astype(jnp.float32)
        pend += [rd_r, rd_l]
    for rd in pend:
        rd.wait_send()

    toff = lax.rem(p + 1, _P) * PC
    boff = half + lax.rem(p - 1 + _P, _P) * PC

    HC = PC // 2
    QC = PC // 4

    def exchange(partner, idx, stgT, stgB_, comT, comB_, t_send, b_send, t_keep, b_keep, size):
        stgT[...] = o_ref[pl.ds(t_send, size)].astype(jnp.bfloat16)
        stgB_[...] = o_ref[pl.ds(b_send, size)].astype(jnp.bfloat16)
        rdt = pltpu.make_async_remote_copy(
            src_ref=stgT,
            dst_ref=comT,
            send_sem=semB_send.at[2 * idx],
            recv_sem=semB_recv.at[2 * idx],
            **dev(partner),
        )
        rdb = pltpu.make_async_remote_copy(
            src_ref=stgB_,
            dst_ref=comB_,
            send_sem=semB_send.at[2 * idx + 1],
            recv_sem=semB_recv.at[2 * idx + 1],
            **dev(partner),
        )
        rdt.start()
        rdb.start()
        return rdt, rdb

    kt1 = toff + bit0 * HC
    kb1 = boff + bit0 * HC
    rdt, rdb = exchange(
        q1, 0, stg1T, stg1B, com1T, com1B,
        toff + (1 - bit0) * HC, boff + (1 - bit0) * HC, kt1, kb1, HC,
    )
    rdt.wait_recv()
    sl = pl.ds(kt1, HC)
    o_ref[sl] = o_ref[sl] + com1T[...].astype(jnp.float32)
    rdb.wait_recv()
    sl = pl.ds(kb1, HC)
    o_ref[sl] = o_ref[sl] + com1B[...].astype(jnp.float32)
    rdt.wait_send()
    rdb.wait_send()

    kt2 = kt1 + bit1 * QC
    kb2 = kb1 + bit1 * QC
    rdt, rdb = exchange(
        q2, 1, stg2T, stg2B, com2T, com2B,
        kt1 + (1 - bit1) * QC, kb1 + (1 - bit1) * QC, kt2, kb2, QC,
    )
    rdt.wait_recv()
    sl = pl.ds(kt2, QC)
    o_ref[sl] = o_ref[sl] + com2T[...].astype(jnp.float32)
    rdb.wait_recv()
    sl = pl.ds(kb2, QC)
    o_ref[sl] = o_ref[sl] + com2B[...].astype(jnp.float32)
    rdt.wait_send()
    rdb.wait_send()

    rdt, rdb = exchange(
        q2, 2, stg3T, stg3B, com3T, com3B, kt2, kb2, 0, 0, QC,
    )
    rdt.wait_recv()
    o_ref[pl.ds(kt1 + (1 - bit1) * QC, QC)] = com3T[...].astype(jnp.float32)
    rdb.wait_recv()
    o_ref[pl.ds(kb1 + (1 - bit1) * QC, QC)] = com3B[...].astype(jnp.float32)
    rdt.wait_send()
    rdb.wait_send()

    rdt, rdb = exchange(
        q1, 3, stg4T, stg4B, com4T, com4B, kt1, kb1, 0, 0, HC,
    )
    rdt.wait_recv()
    o_ref[pl.ds(toff + (1 - bit0) * HC, HC)] = com4T[...].astype(jnp.float32)
    rdb.wait_recv()
    o_ref[pl.ds(boff + (1 - bit0) * HC, HC)] = com4B[...].astype(jnp.float32)
    rdt.wait_send()
    rdb.wait_send()

    stgC_r[...] = o_ref[pl.ds(toff, PC)].astype(jnp.bfloat16)
    stgC_l[...] = o_ref[pl.ds(boff, PC)].astype(jnp.bfloat16)
    pend = []
    for s in range(_P - 1):
        src_r = stgC_r if s == 0 else comC_r.at[s - 1]
        rd_r = pltpu.make_async_remote_copy(
            src_ref=src_r,
            dst_ref=comC_r.at[s],
            send_sem=semC_send_r.at[s],
            recv_sem=semC_recv_r.at[s],
            **dev(right),
        )
        src_l = stgC_l if s == 0 else comC_l.at[s - 1]
        rd_l = pltpu.make_async_remote_copy(
            src_ref=src_l,
            dst_ref=comC_l.at[s],
            send_sem=semC_send_l.at[s],
            recv_sem=semC_recv_l.at[s],
            **dev(left),
        )
        rd_r.start()
        rd_l.start()
        rd_r.wait_recv()
        o_ref[pl.ds(lax.rem(p - s + _P, _P) * PC, PC)] = comC_r[s].astype(
            jnp.float32
        )
        rd_l.wait_recv()
        o_ref[pl.ds(half + lax.rem(p + s, _P) * PC, PC)] = comC_l[s].astype(
            jnp.float32
        )
        pend += [rd_r, rd_l]
    for rd in pend:
        rd.wait_send()


def _ring_allreduce(x2d, cid):
    rows, cols = x2d.shape
    PC = rows // 2 // _P
    HC, QC = PC // 2, PC // 4
    bf = jnp.bfloat16
    bufs = [
        pltpu.VMEM((_P - 1, PC, cols), bf),
        pltpu.VMEM((_P - 1, PC, cols), bf),
        pltpu.VMEM((_P - 1, PC, cols), bf),
        pltpu.VMEM((_P - 1, PC, cols), bf),
        pltpu.VMEM((HC, cols), bf),
        pltpu.VMEM((HC, cols), bf),
        pltpu.VMEM((HC, cols), bf),
        pltpu.VMEM((HC, cols), bf),
        pltpu.VMEM((QC, cols), bf),
        pltpu.VMEM((QC, cols), bf),
        pltpu.VMEM((QC, cols), bf),
        pltpu.VMEM((QC, cols), bf),
        pltpu.VMEM((QC, cols), bf),
        pltpu.VMEM((QC, cols), bf),
        pltpu.VMEM((QC, cols), bf),
        pltpu.VMEM((QC, cols), bf),
        pltpu.VMEM((HC, cols), bf),
        pltpu.VMEM((HC, cols), bf),
        pltpu.VMEM((HC, cols), bf),
        pltpu.VMEM((HC, cols), bf),
        pltpu.VMEM((PC, cols), bf),
        pltpu.VMEM((_P - 1, PC, cols), bf),
        pltpu.VMEM((PC, cols), bf),
        pltpu.VMEM((_P - 1, PC, cols), bf),
    ]
    sems = [
        pltpu.SemaphoreType.DMA((_P - 1,)),
        pltpu.SemaphoreType.DMA((_P - 1,)),
        pltpu.SemaphoreType.DMA((_P - 1,)),
        pltpu.SemaphoreType.DMA((_P - 1,)),
        pltpu.SemaphoreType.DMA((8,)),
        pltpu.SemaphoreType.DMA((8,)),
        pltpu.SemaphoreType.DMA((_P - 1,)),
        pltpu.SemaphoreType.DMA((_P - 1,)),
        pltpu.SemaphoreType.DMA((_P - 1,)),
        pltpu.SemaphoreType.DMA((_P - 1,)),
    ]
    return pl.pallas_call(
        _hier_allreduce_body,
        out_shape=jax.ShapeDtypeStruct((rows, cols), jnp.float32),
        in_specs=[pl.BlockSpec(memory_space=pltpu.VMEM)],
        out_specs=pl.BlockSpec(memory_space=pltpu.VMEM),
        scratch_shapes=bufs + sems,
        compiler_params=pltpu.CompilerParams(collective_id=cid),
    )(x2d)


def kernel(x, Wq, Wk, Wv, Wo, t_emb, W_mod, W_ff1, W_ff2):
    mod = jnp.dot(t_emb, W_mod)
    sa, sha, ga, sm, shm, gm = jnp.split(mod, 6, axis=-1)

    xn1 = _ln_mod(x, sa, sha)
    attn_p = _attn_partial(xn1, Wq, Wk, Wv, Wo)
    attn_sum = _ring_allreduce(attn_p.reshape(B * S, D), 0).reshape(B, S, D)
    x1 = _resid(x, ga, attn_sum)

    xn2 = _ln_mod(x1, sm, shm)
    mlp_p = _mlp_partial(xn2, W_ff1, W_ff2)
    mlp_sum = _ring_allreduce(mlp_p.reshape(B * S, D), 1).reshape(B, S, D)
    return _resid(x1, gm, mlp_sum)
